# baseline (speedup 1.0000x reference)
import jax
import jax.numpy as jnp
from jax.experimental import pallas as pl
from jax.experimental.pallas import tpu as pltpu

GNN_STEPS = 8
H = 512
NUM_CLASSES = 2
NUM_GRAPHS = 64
N_NODES = 10000
ROWS = 10240
RB = 512
N_BLOCKS = ROWS // RB


def _matmul_body(h_ref, w_ref, o_ref):
    o_ref[...] = jnp.dot(h_ref[...], w_ref[...],
                         preferred_element_type=jnp.float32)


def _msg_matmul(h, w):
    return pl.pallas_call(
        _matmul_body,
        grid=(N_BLOCKS,),
        in_specs=[
            pl.BlockSpec((RB, H), lambda i: (i, 0)),
            pl.BlockSpec((H, H), lambda i: (0, 0)),
        ],
        out_specs=pl.BlockSpec((RB, H), lambda i: (i, 0)),
        out_shape=jax.ShapeDtypeStruct((ROWS, H), jnp.float32),
    )(h, w)


def _dot_bias_body(a_ref, w_ref, b_ref, o_ref):
    o_ref[...] = jnp.dot(a_ref[...], w_ref[...],
                         preferred_element_type=jnp.float32) + b_ref[...]


def _dot_bias(a, wT, b2):
    return pl.pallas_call(
        _dot_bias_body,
        grid=(N_BLOCKS,),
        in_specs=[
            pl.BlockSpec((RB, H), lambda i: (i, 0)),
            pl.BlockSpec((H, 3 * H), lambda i: (0, 0)),
            pl.BlockSpec((1, 3 * H), lambda i: (0, 0)),
        ],
        out_specs=pl.BlockSpec((RB, 3 * H), lambda i: (i, 0)),
        out_shape=jax.ShapeDtypeStruct((ROWS, 3 * H), jnp.float32),
    )(a, wT, b2)


def _gates_body(gi_ref, gh_ref, h_ref, o_ref):
    gi, gh = gi_ref[...], gh_ref[...]
    i_r, i_z, i_n = gi[:, :H], gi[:, H:2 * H], gi[:, 2 * H:]
    h_r, h_z, h_n = gh[:, :H], gh[:, H:2 * H], gh[:, 2 * H:]
    r = jax.nn.sigmoid(i_r + h_r)
    z = jax.nn.sigmoid(i_z + h_z)
    n = jnp.tanh(i_n + r * h_n)
    o_ref[...] = (1.0 - z) * n + z * h_ref[...]


def _gates(gi, gh, h):
    return pl.pallas_call(
        _gates_body,
        grid=(N_BLOCKS,),
        in_specs=[
            pl.BlockSpec((RB, 3 * H), lambda i: (i, 0)),
            pl.BlockSpec((RB, 3 * H), lambda i: (i, 0)),
            pl.BlockSpec((RB, H), lambda i: (i, 0)),
        ],
        out_specs=pl.BlockSpec((RB, H), lambda i: (i, 0)),
        out_shape=jax.ShapeDtypeStruct((ROWS, H), jnp.float32),
    )(gi, gh, h)


def _pool_body(h_ref, b_ref, o_ref):
    @pl.when(pl.program_id(0) == 0)
    def _():
        o_ref[...] = jnp.zeros_like(o_ref)
    seg = b_ref[...]
    ids = jax.lax.broadcasted_iota(jnp.int32, (1, NUM_GRAPHS), 1)
    onehot = (seg == ids).astype(jnp.float32)
    o_ref[...] += jax.lax.dot_general(
        onehot, h_ref[...], (((0,), (0,)), ((), ())),
        preferred_element_type=jnp.float32,
        precision=jax.lax.Precision.HIGHEST)


def _pool(h, batch2):
    return pl.pallas_call(
        _pool_body,
        grid=(N_BLOCKS,),
        in_specs=[
            pl.BlockSpec((RB, H), lambda i: (i, 0)),
            pl.BlockSpec((RB, 1), lambda i: (i, 0)),
        ],
        out_specs=pl.BlockSpec((NUM_GRAPHS, H), lambda i: (0, 0)),
        out_shape=jax.ShapeDtypeStruct((NUM_GRAPHS, H), jnp.float32),
    )(h, batch2)


def _mlp_body(g_ref, w1_ref, b1_ref, w2_ref, b2_ref, w3_ref, b3_ref,
              wc_ref, bc_ref, o_ref):
    f = jax.nn.relu(jnp.dot(g_ref[...], w1_ref[...],
                            preferred_element_type=jnp.float32) + b1_ref[...])
    f = jax.nn.relu(jnp.dot(f, w2_ref[...],
                            preferred_element_type=jnp.float32) + b2_ref[...])
    f = jax.nn.relu(jnp.dot(f, w3_ref[...],
                            preferred_element_type=jnp.float32) + b3_ref[...])
    o_ref[...] = jnp.dot(f, wc_ref[...],
                         preferred_element_type=jnp.float32) + bc_ref[...]


def _mlp(g, w1T, b1, w2T, b2, w3T, b3, wcT, bc):
    return pl.pallas_call(
        _mlp_body,
        out_shape=jax.ShapeDtypeStruct((NUM_GRAPHS, 128), jnp.float32),
    )(g, w1T, b1, w2T, b2, w3T, b3, wcT, bc)


def kernel(x, edge_index, batch, weight, w_ih, w_hh, b_ih, b_hh,
           W1, b1, W2, b2, W3, b3, Wc, bc):
    src = edge_index[0]
    dst = edge_index[1]

    h = jnp.zeros((ROWS, H), jnp.float32).at[:N_NODES, :x.shape[1]].set(x)

    wihT = w_ih.T
    whhT = w_hh.T
    bih2 = b_ih.reshape(1, 3 * H)
    bhh2 = b_hh.reshape(1, 3 * H)

    for i in range(GNN_STEPS):
        m = _msg_matmul(h, weight[i])
        agg = jax.ops.segment_sum(m[:N_NODES][src], dst, num_segments=N_NODES)
        agg = jnp.zeros((ROWS, H), jnp.float32).at[:N_NODES].set(agg)
        gi = _dot_bias(agg, wihT, bih2)
        gh = _dot_bias(h, whhT, bhh2)
        h = _gates(gi, gh, h)

    batch_pad = jnp.full((ROWS, 1), NUM_GRAPHS, jnp.int32).at[:N_NODES, 0].set(batch)
    g = _pool(h, batch_pad)

    wcT = jnp.zeros((2 * H, 128), jnp.float32).at[:, :NUM_CLASSES].set(Wc.T)
    bc_pad = jnp.zeros((1, 128), jnp.float32).at[0, :NUM_CLASSES].set(bc)
    out = _mlp(g, W1.T, b1.reshape(1, -1), W2.T, b2.reshape(1, -1),
               W3.T, b3.reshape(1, -1), wcT, bc_pad)
    return out[:, :NUM_CLASSES]


# speedup vs baseline: 1.2893x; 1.2893x over previous
import functools

import jax
import jax.numpy as jnp
from jax import lax
from jax.experimental import pallas as pl
from jax.experimental.pallas import tpu as pltpu
from jax.experimental.pallas import tpu_sc as plsc

GNN_STEPS = 8
H = 512
NUM_CLASSES = 2
NUM_GRAPHS = 64
N_NODES = 10000
ROWS = 10240
RB = 512
N_BLOCKS = ROWS // RB

NW = 32
KCH = 64
NRANGE = 64
NR_PER_W = NRANGE // NW
RNG = ROWS // NRANGE
ACC_ROWS = RNG + 8
N_EDGES = 160000
CAP_CH = N_EDGES // KCH + NRANGE
CAP_E = CAP_CH * KCH


def _seg_body(m_hbm, srcp_hbm, dstp_hbm, nch_hbm, cst_hbm, agg_hbm,
              nch_v, cst_v, sidx, didx, gbuf, acc, sem):
    wid = lax.axis_index("s") * 2 + lax.axis_index("c")
    pltpu.sync_copy(nch_hbm, nch_v)
    pltpu.sync_copy(cst_hbm, cst_v)
    zero16 = jnp.zeros((16,), jnp.float32)

    for j in range(NR_PER_W):
        r = wid * NR_PER_W + j

        def pick(tbl):
            return tbl[pl.ds(r, 16)][0]

        n_c = pick(nch_v)
        c_s = pick(cst_v)

        def zb(i, carry):
            for cc in range(H // 16):
                acc[i, pl.ds(cc * 16, 16)] = zero16
            return carry

        lax.fori_loop(0, ACC_ROWS, zb, 0)

        def cb(c, carry):
            e0 = pl.multiple_of((c_s + c) * KCH, KCH)
            pltpu.sync_copy(srcp_hbm.at[pl.ds(e0, KCH)], sidx)
            pltpu.sync_copy(dstp_hbm.at[pl.ds(e0, KCH)],
                            didx.at[pl.ds(0, KCH)])
            pltpu.async_copy(m_hbm.at[sidx], gbuf, sem).wait()

            def eb(e, carry2):
                d = didx[pl.ds(e, 16)][0]
                for c2 in range(H // 16):
                    sl = pl.ds(c2 * 16, 16)
                    acc[d, sl] = acc[d, sl] + gbuf[e, sl]
                return carry2

            lax.fori_loop(0, KCH, eb, 0)
            return carry

        lax.fori_loop(0, n_c, cb, 0)
        base = pl.multiple_of(r * RNG, RNG)
        pltpu.sync_copy(acc.at[pl.ds(0, RNG)], agg_hbm.at[pl.ds(base, RNG)])


@jax.jit
def _seg_sum_sc(m, src_p, dst_p, nch, cst):
    mesh = plsc.VectorSubcoreMesh(core_axis_name="c", subcore_axis_name="s")
    return pl.kernel(
        _seg_body,
        out_type=jax.ShapeDtypeStruct((ROWS, H), jnp.float32),
        mesh=mesh,
        scratch_types=[
            pltpu.VMEM((NRANGE + 16,), jnp.int32),
            pltpu.VMEM((NRANGE + 16,), jnp.int32),
            pltpu.VMEM((KCH,), jnp.int32),
            pltpu.VMEM((KCH + 16,), jnp.int32),
            pltpu.VMEM((KCH, H), jnp.float32),
            pltpu.VMEM((ACC_ROWS, H), jnp.float32),
            pltpu.SemaphoreType.DMA,
        ],
    )(m, src_p, dst_p, nch, cst)


def _edge_plan(src, dst):
    perm = jnp.argsort(dst, stable=True)
    src_s = src[perm]
    dst_s = dst[perm]
    rid = dst_s // RNG
    cnt = jnp.zeros((NRANGE,), jnp.int32).at[rid].add(1)
    nch = (cnt + KCH - 1) // KCH
    cst = jnp.concatenate([jnp.zeros((1,), jnp.int32), jnp.cumsum(nch)[:-1]])
    estart = jnp.concatenate([jnp.zeros((1,), jnp.int32), jnp.cumsum(cnt)[:-1]])
    eidx = jnp.arange(N_EDGES, dtype=jnp.int32)
    pos = cst[rid] * KCH + (eidx - estart[rid])
    fill = jnp.arange(CAP_E, dtype=jnp.int32)
    src_p = (N_NODES + fill % (ROWS - N_NODES)).at[pos].set(src_s)
    dst_p = (RNG + fill % (ACC_ROWS - RNG)).at[pos].set(dst_s - rid * RNG)
    return (src_p.astype(jnp.int32), dst_p.astype(jnp.int32),
            jnp.pad(nch, (0, 16)).astype(jnp.int32),
            jnp.pad(cst, (0, 16)).astype(jnp.int32))


def _matmul_body(h_ref, w_ref, o_ref):
    o_ref[...] = jnp.dot(h_ref[...], w_ref[...],
                         preferred_element_type=jnp.float32)


def _msg_matmul(h, w):
    return pl.pallas_call(
        _matmul_body,
        grid=(N_BLOCKS,),
        in_specs=[
            pl.BlockSpec((RB, H), lambda i: (i, 0)),
            pl.BlockSpec((H, H), lambda i: (0, 0)),
        ],
        out_specs=pl.BlockSpec((RB, H), lambda i: (i, 0)),
        out_shape=jax.ShapeDtypeStruct((ROWS, H), jnp.float32),
    )(h, w)


def _dot_bias_body(a_ref, w_ref, b_ref, o_ref):
    o_ref[...] = jnp.dot(a_ref[...], w_ref[...],
                         preferred_element_type=jnp.float32) + b_ref[...]


def _dot_bias(a, wT, b2):
    return pl.pallas_call(
        _dot_bias_body,
        grid=(N_BLOCKS,),
        in_specs=[
            pl.BlockSpec((RB, H), lambda i: (i, 0)),
            pl.BlockSpec((H, 3 * H), lambda i: (0, 0)),
            pl.BlockSpec((1, 3 * H), lambda i: (0, 0)),
        ],
        out_specs=pl.BlockSpec((RB, 3 * H), lambda i: (i, 0)),
        out_shape=jax.ShapeDtypeStruct((ROWS, 3 * H), jnp.float32),
    )(a, wT, b2)


def _gates_body(gi_ref, gh_ref, h_ref, o_ref):
    gi, gh = gi_ref[...], gh_ref[...]
    i_r, i_z, i_n = gi[:, :H], gi[:, H:2 * H], gi[:, 2 * H:]
    h_r, h_z, h_n = gh[:, :H], gh[:, H:2 * H], gh[:, 2 * H:]
    r = jax.nn.sigmoid(i_r + h_r)
    z = jax.nn.sigmoid(i_z + h_z)
    n = jnp.tanh(i_n + r * h_n)
    o_ref[...] = (1.0 - z) * n + z * h_ref[...]


def _gates(gi, gh, h):
    return pl.pallas_call(
        _gates_body,
        grid=(N_BLOCKS,),
        in_specs=[
            pl.BlockSpec((RB, 3 * H), lambda i: (i, 0)),
            pl.BlockSpec((RB, 3 * H), lambda i: (i, 0)),
            pl.BlockSpec((RB, H), lambda i: (i, 0)),
        ],
        out_specs=pl.BlockSpec((RB, H), lambda i: (i, 0)),
        out_shape=jax.ShapeDtypeStruct((ROWS, H), jnp.float32),
    )(gi, gh, h)


def _pool_body(h_ref, b_ref, o_ref):
    @pl.when(pl.program_id(0) == 0)
    def _():
        o_ref[...] = jnp.zeros_like(o_ref)
    seg = b_ref[...]
    ids = jax.lax.broadcasted_iota(jnp.int32, (1, NUM_GRAPHS), 1)
    onehot = (seg == ids).astype(jnp.float32)
    o_ref[...] += jax.lax.dot_general(
        onehot, h_ref[...], (((0,), (0,)), ((), ())),
        preferred_element_type=jnp.float32,
        precision=jax.lax.Precision.HIGHEST)


def _pool(h, batch2):
    return pl.pallas_call(
        _pool_body,
        grid=(N_BLOCKS,),
        in_specs=[
            pl.BlockSpec((RB, H), lambda i: (i, 0)),
            pl.BlockSpec((RB, 1), lambda i: (i, 0)),
        ],
        out_specs=pl.BlockSpec((NUM_GRAPHS, H), lambda i: (0, 0)),
        out_shape=jax.ShapeDtypeStruct((NUM_GRAPHS, H), jnp.float32),
    )(h, batch2)


def _mlp_body(g_ref, w1_ref, b1_ref, w2_ref, b2_ref, w3_ref, b3_ref,
              wc_ref, bc_ref, o_ref):
    f = jax.nn.relu(jnp.dot(g_ref[...], w1_ref[...],
                            preferred_element_type=jnp.float32) + b1_ref[...])
    f = jax.nn.relu(jnp.dot(f, w2_ref[...],
                            preferred_element_type=jnp.float32) + b2_ref[...])
    f = jax.nn.relu(jnp.dot(f, w3_ref[...],
                            preferred_element_type=jnp.float32) + b3_ref[...])
    o_ref[...] = jnp.dot(f, wc_ref[...],
                         preferred_element_type=jnp.float32) + bc_ref[...]


def _mlp(g, w1T, b1, w2T, b2, w3T, b3, wcT, bc):
    return pl.pallas_call(
        _mlp_body,
        out_shape=jax.ShapeDtypeStruct((NUM_GRAPHS, 128), jnp.float32),
    )(g, w1T, b1, w2T, b2, w3T, b3, wcT, bc)


def kernel(x, edge_index, batch, weight, w_ih, w_hh, b_ih, b_hh,
           W1, b1, W2, b2, W3, b3, Wc, bc):
    src = edge_index[0]
    dst = edge_index[1]

    h = jnp.zeros((ROWS, H), jnp.float32).at[:N_NODES, :x.shape[1]].set(x)

    wihT = w_ih.T
    whhT = w_hh.T
    bih2 = b_ih.reshape(1, 3 * H)
    bhh2 = b_hh.reshape(1, 3 * H)

    src_p, dst_p, nch, cst = _edge_plan(src, dst)

    for i in range(GNN_STEPS):
        m = _msg_matmul(h, weight[i])
        agg = _seg_sum_sc(m, src_p, dst_p, nch, cst)
        gi = _dot_bias(agg, wihT, bih2)
        gh = _dot_bias(h, whhT, bhh2)
        h = _gates(gi, gh, h)

    batch_pad = jnp.full((ROWS, 1), NUM_GRAPHS, jnp.int32).at[:N_NODES, 0].set(batch)
    g = _pool(h, batch_pad)

    wcT = jnp.zeros((2 * H, 128), jnp.float32).at[:, :NUM_CLASSES].set(Wc.T)
    bc_pad = jnp.zeros((1, 128), jnp.float32).at[0, :NUM_CLASSES].set(bc)
    out = _mlp(g, W1.T, b1.reshape(1, -1), W2.T, b2.reshape(1, -1),
               W3.T, b3.reshape(1, -1), wcT, bc_pad)
    return out[:, :NUM_CLASSES]


# speedup vs baseline: 1.3757x; 1.0671x over previous
import functools

import jax
import jax.numpy as jnp
from jax import lax
from jax.experimental import pallas as pl
from jax.experimental.pallas import tpu as pltpu
from jax.experimental.pallas import tpu_sc as plsc

GNN_STEPS = 8
H = 512
NUM_CLASSES = 2
NUM_GRAPHS = 64
N_NODES = 10000
ROWS = 10240
RB = 512
N_BLOCKS = ROWS // RB

NW = 32
KCH = 32
NRANGE = 64
NR_PER_W = NRANGE // NW
RNG = ROWS // NRANGE
ACC_ROWS = RNG + 8
N_EDGES = 160000
CAP_CH = N_EDGES // KCH + 2 * NRANGE
CAP_E = CAP_CH * KCH


def _seg_body(m_hbm, srcp_hbm, dstp_hbm, nch_hbm, cst_hbm, agg_hbm,
              nch_v, cst_v, sidx0, didx0, gbuf0, sidx1, didx1, gbuf1,
              acc, sem0, sem1):
    wid = lax.axis_index("s") * 2 + lax.axis_index("c")
    pltpu.sync_copy(nch_hbm, nch_v)
    pltpu.sync_copy(cst_hbm, cst_v)
    zero16 = jnp.zeros((16,), jnp.float32)

    bufs = ((sidx0, didx0, gbuf0, sem0), (sidx1, didx1, gbuf1, sem1))

    for j in range(NR_PER_W):
        r = wid * NR_PER_W + j

        def pick(tbl):
            return tbl[pl.ds(r, 16)][0]

        n_half = pick(nch_v) // 2
        c_s = pick(cst_v)

        def zb(i, carry):
            for cc in range(H // 16):
                acc[i, pl.ds(cc * 16, 16)] = zero16
            return carry

        lax.fori_loop(0, ACC_ROWS, zb, 0)

        def fire(q, b):
            sidx, didx, gbuf, sem = bufs[b]
            e0 = pl.multiple_of((c_s + q) * KCH, KCH)
            pltpu.sync_copy(srcp_hbm.at[pl.ds(e0, KCH)], sidx)
            pltpu.sync_copy(dstp_hbm.at[pl.ds(e0, KCH)],
                            didx.at[pl.ds(0, KCH)])
            return pltpu.async_copy(m_hbm.at[sidx], gbuf, sem)

        def drain_and_accum(b):
            sidx, didx, gbuf, sem = bufs[b]
            pltpu.make_async_copy(m_hbm.at[sidx], gbuf, sem).wait()

            def eb(e, carry2):
                d = didx[pl.ds(e, 16)][0]
                for c2 in range(H // 16):
                    sl = pl.ds(c2 * 16, 16)
                    acc[d, sl] = acc[d, sl] + gbuf[e, sl]
                return carry2

            lax.fori_loop(0, KCH, eb, 0)

        fire(0, 0)
        fire(1, 1)

        def pair_body(k, carry):
            c = k * 2
            drain_and_accum(0)

            @pl.when(k < n_half - 1)
            def _():
                fire(c + 2, 0)

            drain_and_accum(1)

            @pl.when(k < n_half - 1)
            def _():
                fire(c + 3, 1)

            return carry

        lax.fori_loop(0, n_half, pair_body, 0)
        base = pl.multiple_of(r * RNG, RNG)
        pltpu.sync_copy(acc.at[pl.ds(0, RNG)], agg_hbm.at[pl.ds(base, RNG)])


@jax.jit
def _seg_sum_sc(m, src_p, dst_p, nch, cst):
    mesh = plsc.VectorSubcoreMesh(core_axis_name="c", subcore_axis_name="s")
    return pl.kernel(
        _seg_body,
        out_type=jax.ShapeDtypeStruct((ROWS, H), jnp.float32),
        mesh=mesh,
        scratch_types=[
            pltpu.VMEM((NRANGE + 16,), jnp.int32),
            pltpu.VMEM((NRANGE + 16,), jnp.int32),
            pltpu.VMEM((KCH,), jnp.int32),
            pltpu.VMEM((KCH + 16,), jnp.int32),
            pltpu.VMEM((KCH, H), jnp.float32),
            pltpu.VMEM((KCH,), jnp.int32),
            pltpu.VMEM((KCH + 16,), jnp.int32),
            pltpu.VMEM((KCH, H), jnp.float32),
            pltpu.VMEM((ACC_ROWS, H), jnp.float32),
            pltpu.SemaphoreType.DMA,
            pltpu.SemaphoreType.DMA,
        ],
    )(m, src_p, dst_p, nch, cst)


def _edge_plan(src, dst):
    perm = jnp.argsort(dst, stable=True)
    src_s = src[perm]
    dst_s = dst[perm]
    rid = dst_s // RNG
    cnt = jnp.zeros((NRANGE,), jnp.int32).at[rid].add(1)
    nch = 2 * jnp.maximum(1, (cnt + 2 * KCH - 1) // (2 * KCH))
    cst = jnp.concatenate([jnp.zeros((1,), jnp.int32), jnp.cumsum(nch)[:-1]])
    estart = jnp.concatenate([jnp.zeros((1,), jnp.int32), jnp.cumsum(cnt)[:-1]])
    eidx = jnp.arange(N_EDGES, dtype=jnp.int32)
    pos = cst[rid] * KCH + (eidx - estart[rid])
    fill = jnp.arange(CAP_E, dtype=jnp.int32)
    src_p = (N_NODES + fill % (ROWS - N_NODES)).at[pos].set(src_s)
    dst_p = (RNG + fill % (ACC_ROWS - RNG)).at[pos].set(dst_s - rid * RNG)
    return (src_p.astype(jnp.int32), dst_p.astype(jnp.int32),
            jnp.pad(nch, (0, 16)).astype(jnp.int32),
            jnp.pad(cst, (0, 16)).astype(jnp.int32))


def _matmul_body(h_ref, w_ref, o_ref):
    o_ref[...] = jnp.dot(h_ref[...], w_ref[...],
                         preferred_element_type=jnp.float32)


def _msg_matmul(h, w):
    return pl.pallas_call(
        _matmul_body,
        grid=(N_BLOCKS,),
        in_specs=[
            pl.BlockSpec((RB, H), lambda i: (i, 0)),
            pl.BlockSpec((H, H), lambda i: (0, 0)),
        ],
        out_specs=pl.BlockSpec((RB, H), lambda i: (i, 0)),
        out_shape=jax.ShapeDtypeStruct((ROWS, H), jnp.float32),
    )(h, w)


def _dot_bias_body(a_ref, w_ref, b_ref, o_ref):
    o_ref[...] = jnp.dot(a_ref[...], w_ref[...],
                         preferred_element_type=jnp.float32) + b_ref[...]


def _dot_bias(a, wT, b2):
    return pl.pallas_call(
        _dot_bias_body,
        grid=(N_BLOCKS,),
        in_specs=[
            pl.BlockSpec((RB, H), lambda i: (i, 0)),
            pl.BlockSpec((H, 3 * H), lambda i: (0, 0)),
            pl.BlockSpec((1, 3 * H), lambda i: (0, 0)),
        ],
        out_specs=pl.BlockSpec((RB, 3 * H), lambda i: (i, 0)),
        out_shape=jax.ShapeDtypeStruct((ROWS, 3 * H), jnp.float32),
    )(a, wT, b2)


def _gates_body(gi_ref, gh_ref, h_ref, o_ref):
    gi, gh = gi_ref[...], gh_ref[...]
    i_r, i_z, i_n = gi[:, :H], gi[:, H:2 * H], gi[:, 2 * H:]
    h_r, h_z, h_n = gh[:, :H], gh[:, H:2 * H], gh[:, 2 * H:]
    r = jax.nn.sigmoid(i_r + h_r)
    z = jax.nn.sigmoid(i_z + h_z)
    n = jnp.tanh(i_n + r * h_n)
    o_ref[...] = (1.0 - z) * n + z * h_ref[...]


def _gates(gi, gh, h):
    return pl.pallas_call(
        _gates_body,
        grid=(N_BLOCKS,),
        in_specs=[
            pl.BlockSpec((RB, 3 * H), lambda i: (i, 0)),
            pl.BlockSpec((RB, 3 * H), lambda i: (i, 0)),
            pl.BlockSpec((RB, H), lambda i: (i, 0)),
        ],
        out_specs=pl.BlockSpec((RB, H), lambda i: (i, 0)),
        out_shape=jax.ShapeDtypeStruct((ROWS, H), jnp.float32),
    )(gi, gh, h)


def _pool_body(h_ref, b_ref, o_ref):
    @pl.when(pl.program_id(0) == 0)
    def _():
        o_ref[...] = jnp.zeros_like(o_ref)
    seg = b_ref[...]
    ids = jax.lax.broadcasted_iota(jnp.int32, (1, NUM_GRAPHS), 1)
    onehot = (seg == ids).astype(jnp.float32)
    o_ref[...] += jax.lax.dot_general(
        onehot, h_ref[...], (((0,), (0,)), ((), ())),
        preferred_element_type=jnp.float32,
        precision=jax.lax.Precision.HIGHEST)


def _pool(h, batch2):
    return pl.pallas_call(
        _pool_body,
        grid=(N_BLOCKS,),
        in_specs=[
            pl.BlockSpec((RB, H), lambda i: (i, 0)),
            pl.BlockSpec((RB, 1), lambda i: (i, 0)),
        ],
        out_specs=pl.BlockSpec((NUM_GRAPHS, H), lambda i: (0, 0)),
        out_shape=jax.ShapeDtypeStruct((NUM_GRAPHS, H), jnp.float32),
    )(h, batch2)


def _mlp_body(g_ref, w1_ref, b1_ref, w2_ref, b2_ref, w3_ref, b3_ref,
              wc_ref, bc_ref, o_ref):
    f = jax.nn.relu(jnp.dot(g_ref[...], w1_ref[...],
                            preferred_element_type=jnp.float32) + b1_ref[...])
    f = jax.nn.relu(jnp.dot(f, w2_ref[...],
                            preferred_element_type=jnp.float32) + b2_ref[...])
    f = jax.nn.relu(jnp.dot(f, w3_ref[...],
                            preferred_element_type=jnp.float32) + b3_ref[...])
    o_ref[...] = jnp.dot(f, wc_ref[...],
                         preferred_element_type=jnp.float32) + bc_ref[...]


def _mlp(g, w1T, b1, w2T, b2, w3T, b3, wcT, bc):
    return pl.pallas_call(
        _mlp_body,
        out_shape=jax.ShapeDtypeStruct((NUM_GRAPHS, 128), jnp.float32),
    )(g, w1T, b1, w2T, b2, w3T, b3, wcT, bc)


def kernel(x, edge_index, batch, weight, w_ih, w_hh, b_ih, b_hh,
           W1, b1, W2, b2, W3, b3, Wc, bc):
    src = edge_index[0]
    dst = edge_index[1]

    h = jnp.zeros((ROWS, H), jnp.float32).at[:N_NODES, :x.shape[1]].set(x)

    wihT = w_ih.T
    whhT = w_hh.T
    bih2 = b_ih.reshape(1, 3 * H)
    bhh2 = b_hh.reshape(1, 3 * H)

    src_p, dst_p, nch, cst = _edge_plan(src, dst)

    for i in range(GNN_STEPS):
        m = _msg_matmul(h, weight[i])
        agg = _seg_sum_sc(m, src_p, dst_p, nch, cst)
        gi = _dot_bias(agg, wihT, bih2)
        gh = _dot_bias(h, whhT, bhh2)
        h = _gates(gi, gh, h)

    batch_pad = jnp.full((ROWS, 1), NUM_GRAPHS, jnp.int32).at[:N_NODES, 0].set(batch)
    g = _pool(h, batch_pad)

    wcT = jnp.zeros((2 * H, 128), jnp.float32).at[:, :NUM_CLASSES].set(Wc.T)
    bc_pad = jnp.zeros((1, 128), jnp.float32).at[0, :NUM_CLASSES].set(bc)
    out = _mlp(g, W1.T, b1.reshape(1, -1), W2.T, b2.reshape(1, -1),
               W3.T, b3.reshape(1, -1), wcT, bc_pad)
    return out[:, :NUM_CLASSES]
